# Initial kernel scaffold; baseline (speedup 1.0000x reference)
import jax
import jax.numpy as jnp
from jax.experimental import pallas as pl


def kernel(x, token_table, pos_table):
    raise NotImplementedError("write your pallas kernel here")


# speedup vs baseline: 2.6792x; 2.6792x over previous
import jax
import jax.numpy as jnp
from jax import lax
from jax.experimental import pallas as pl
from jax.experimental.pallas import tpu as pltpu
from jax.experimental.pallas import tpu_sc as plsc

B, S, D = 1024, 200, 128
N = B * S
NC, NS = 2, 16
NW = NC * NS
ROWS_PER_W = N // NW
CHUNK = 40
NCHUNK = ROWS_PER_W // CHUNK
NGROUP = NCHUNK // 5


def _sc_embed(xf, token_table, pos_table):
    mesh = plsc.VectorSubcoreMesh(core_axis_name="c", subcore_axis_name="s")

    @pl.kernel(
        out_type=jax.ShapeDtypeStruct((N, D), jnp.float32),
        mesh=mesh,
        scratch_types=[
            pltpu.VMEM((NCHUNK, CHUNK), jnp.int32),
            pltpu.VMEM((S, D), jnp.float32),
            pltpu.VMEM((5, CHUNK, D), jnp.float32),
            pltpu.SemaphoreType.DMA,
        ],
    )
    def k(xf_hbm, tok_hbm, pos_hbm, out_hbm, idx_v, pos_v, bufs, sem):
        wid = lax.axis_index("s") * NC + lax.axis_index("c")
        base_row = wid * ROWS_PER_W
        pltpu.sync_copy(xf_hbm.at[wid], idx_v)
        pltpu.sync_copy(pos_hbm, pos_v)

        def step(t, _):
            for p in range(5):
                j = 5 * t + p
                buf = bufs.at[p]
                pltpu.async_copy(tok_hbm.at[idx_v.at[j]], buf, sem).wait()

                def add_row(r, _):
                    for k in range(D // 16):
                        c = pl.ds(k * 16, 16)
                        buf[r, c] = buf[r, c] + pos_v[p * CHUNK + r, c]
                    return _

                lax.fori_loop(0, CHUNK, add_row, None)
                pltpu.sync_copy(
                    buf, out_hbm.at[pl.ds(base_row + j * CHUNK, CHUNK)])
            return _

        lax.fori_loop(0, NGROUP, step, None)

    return k(xf, token_table, pos_table)


def kernel(x, token_table, pos_table):
    xf = x.reshape(NW, NCHUNK, CHUNK).astype(jnp.int32)
    out = _sc_embed(xf, token_table, pos_table)
    return out.reshape(B, S, D)


# speedup vs baseline: 7.3172x; 2.7311x over previous
import jax
import jax.numpy as jnp
from jax import lax
from jax.experimental import pallas as pl
from jax.experimental.pallas import tpu as pltpu
from jax.experimental.pallas import tpu_sc as plsc

B, S, D = 1024, 200, 128
N = B * S
NC, NS = 2, 16
NW = NC * NS
ROWS_PER_W = N // NW
CHUNK = 40
NCHUNK = ROWS_PER_W // CHUNK
SLOTS = 10
LEAD = 5


def _sc_embed(xf, token_table, pos_table):
    mesh = plsc.VectorSubcoreMesh(core_axis_name="c", subcore_axis_name="s")

    @pl.kernel(
        out_type=jax.ShapeDtypeStruct((N, D), jnp.float32),
        mesh=mesh,
        scratch_types=[
            pltpu.VMEM((NCHUNK, CHUNK), jnp.int32),
            pltpu.VMEM((S, D), jnp.float32),
            pltpu.VMEM((SLOTS, CHUNK, D), jnp.float32),
            pltpu.SemaphoreType.DMA((SLOTS,)),
            pltpu.SemaphoreType.DMA((SLOTS,)),
        ],
    )
    def k(xf_hbm, tok_hbm, pos_hbm, out_hbm, idx_v, pos_v, bufs, gsem, ssem):
        wid = lax.axis_index("s") * NC + lax.axis_index("c")
        base_row = wid * ROWS_PER_W
        pltpu.sync_copy(xf_hbm.at[wid], idx_v)
        pltpu.sync_copy(pos_hbm, pos_v)

        def gather(j, slot):
            pltpu.async_copy(tok_hbm.at[idx_v.at[j]], bufs.at[slot],
                             gsem.at[slot])

        def consume(j, slot):
            p = slot % 5
            buf = bufs.at[slot]
            pltpu.make_async_copy(tok_hbm.at[idx_v.at[j]], buf,
                                  gsem.at[slot]).wait()

            def add_row(r, _):
                for kk in range(D // 16):
                    c = pl.ds(kk * 16, 16)
                    buf[r, c] = buf[r, c] + pos_v[p * CHUNK + r, c]
                return _

            lax.fori_loop(0, CHUNK, add_row, None)
            pltpu.async_copy(
                buf, out_hbm.at[pl.ds(base_row + j * CHUNK, CHUNK)],
                ssem.at[slot])

        def drain_scatter(slot):
            pltpu.make_async_copy(bufs.at[slot], out_hbm.at[pl.ds(0, CHUNK)],
                                  ssem.at[slot]).wait()

        def refill(j, slot, first):
            s2 = (slot + LEAD) % SLOTS
            if not first:
                drain_scatter(s2)
            gather(j + LEAD, s2)

        for s in range(LEAD):
            gather(s, s)
        for i in range(SLOTS):
            consume(i, i)
            refill(i, i, first=i < LEAD)

        def step(u, _):
            for i in range(SLOTS):
                j = SLOTS + u * SLOTS + i
                consume(j, i)
                refill(j, i, first=False)
            return _

        lax.fori_loop(0, (NCHUNK - 2 * SLOTS) // SLOTS, step, None)

        for i in range(SLOTS):
            j = NCHUNK - SLOTS + i
            consume(j, i)
            if j + LEAD < NCHUNK:
                refill(j, i, first=False)
        for s in range(SLOTS):
            drain_scatter(s)

    return k(xf, token_table, pos_table)


def kernel(x, token_table, pos_table):
    xf = x.reshape(NW, NCHUNK, CHUNK).astype(jnp.int32)
    out = _sc_embed(xf, token_table, pos_table)
    return out.reshape(B, S, D)
